# Initial kernel scaffold; baseline (speedup 1.0000x reference)
import jax
import jax.numpy as jnp
from jax import lax
from jax.experimental import pallas as pl
from jax.experimental.pallas import tpu as pltpu
from jax.experimental.pallas import tpu_sc as plsc

L = 2048
CA = 768
CS = 384
CP = 16
NH = 12
DH = 64
KNN = 128
E = L * KNN


_RA = 512


def _stage_a_body(qL_ref, cL_ref, lnsw_ref, wadag_ref, badag_ref, wadal_ref,
                  wq_ref, wk_ref, wv_ref, wg_ref, wqr_ref, wkr_ref,
                  woutg_ref, boutg_ref,
                  q_out, k_out, v_out, g_out, gate_out):
    f32 = jnp.float32
    x = qL_ref[...]
    mu = jnp.mean(x, axis=1, keepdims=True)
    xc = x - mu
    var = jnp.mean(xc * xc, axis=1, keepdims=True)
    a = xc * lax.rsqrt(var + 1e-5)

    c = cL_ref[...]
    cmu = jnp.mean(c, axis=1, keepdims=True)
    cc = c - cmu
    cvar = jnp.mean(cc * cc, axis=1, keepdims=True)
    s = cc * lax.rsqrt(cvar + 1e-5) * lnsw_ref[...]

    adag = jax.nn.sigmoid(
        jnp.dot(s, wadag_ref[...], preferred_element_type=f32) + badag_ref[...])
    a = adag * a + jnp.dot(s, wadal_ref[...], preferred_element_type=f32)

    q = jnp.dot(a, wq_ref[...], preferred_element_type=f32)
    qms = jnp.mean(q * q, axis=1, keepdims=True)
    q = q * lax.rsqrt(qms + 1e-8) * wqr_ref[...]
    q_out[...] = q * (1.0 / (DH ** 0.5))

    k = jnp.dot(a, wk_ref[...], preferred_element_type=f32)
    kms = jnp.mean(k * k, axis=1, keepdims=True)
    k_out[...] = k * lax.rsqrt(kms + 1e-8) * wkr_ref[...]

    v_out[...] = jnp.dot(a, wv_ref[...], preferred_element_type=f32)
    g_out[...] = jax.nn.sigmoid(
        jnp.dot(a, wg_ref[...], preferred_element_type=f32))
    gate_out[...] = jax.nn.sigmoid(
        jnp.dot(c, woutg_ref[...], preferred_element_type=f32) + boutg_ref[...])


def _stage_a(qL, cL, lnsw, wadagT, badag, wadalT, wqT, wkT, wvT, wgT,
             wqr, wkr, woutgT, boutg):
    f32 = jnp.float32

    def row_spec(w):
        return pl.BlockSpec((_RA, w), lambda i: (i, 0))

    def full(a):
        return pl.BlockSpec(a.shape, lambda i: tuple(0 for _ in a.shape))

    outs = [jax.ShapeDtypeStruct((L, CA), f32) for _ in range(5)]
    return pl.pallas_call(
        _stage_a_body,
        grid=(L // _RA,),
        in_specs=[row_spec(CA), row_spec(CS), full(lnsw), full(wadagT),
                  full(badag), full(wadalT), full(wqT), full(wkT), full(wvT),
                  full(wgT), full(wqr), full(wkr), full(woutgT), full(boutg)],
        out_specs=[row_spec(CA)] * 5,
        out_shape=outs,
    )(qL, cL, lnsw, wadagT, badag, wadalT, wqT, wkT, wvT, wgT, wqr, wkr,
      woutgT, boutg)


_CH = 64


def _sc_gather_body(idx_hbm, k_hbm, v_hbm, p_hbm,
                    kg_hbm, vg_hbm, pg_hbm,
                    idx_v, pidx_v, kbuf, vbuf, pbuf, semk, semv, semp):
    info = plsc.get_sparse_core_info()
    nw = info.num_cores * info.num_subcores
    per_w = E // nw
    n_chunks = per_w // _CH
    wid = lax.axis_index("c") * info.num_subcores + lax.axis_index("s")
    base_w = wid * per_w

    def chunk(t, carry):
        base = base_w + t * _CH
        pltpu.sync_copy(idx_hbm.at[pl.ds(base, _CH)], idx_v)
        for j in range(_CH // 16):
            iv = idx_v[pl.ds(j * 16, 16)]
            e0 = base + j * 16
            row = (lax.iota(jnp.int32, (16,)) + e0) >> 7
            pidx_v[pl.ds(j * 16, 16)] = (row << 11) + iv
        cpk = pltpu.async_copy(k_hbm.at[idx_v], kbuf, semk)
        cpv = pltpu.async_copy(v_hbm.at[idx_v], vbuf, semv)
        cpp = pltpu.async_copy(p_hbm.at[pidx_v], pbuf, semp)
        cpk.wait()
        pltpu.sync_copy(kbuf, kg_hbm.at[pl.ds(base, _CH)])
        cpv.wait()
        pltpu.sync_copy(vbuf, vg_hbm.at[pl.ds(base, _CH)])
        cpp.wait()
        pltpu.sync_copy(pbuf, pg_hbm.at[pl.ds(base, _CH)])
        return carry

    lax.fori_loop(0, n_chunks, chunk, 0)


def _sc_gather(idx_flat, k, v, p2):
    f32 = jnp.float32
    i32 = jnp.int32
    mesh = plsc.VectorSubcoreMesh(core_axis_name="c", subcore_axis_name="s")
    fn = pl.kernel(
        _sc_gather_body,
        out_type=[jax.ShapeDtypeStruct((E, CA), f32),
                  jax.ShapeDtypeStruct((E, CA), f32),
                  jax.ShapeDtypeStruct((E, CP), f32)],
        mesh=mesh,
        scratch_types=[pltpu.VMEM((_CH,), i32), pltpu.VMEM((_CH,), i32),
                       pltpu.VMEM((_CH, CA), f32), pltpu.VMEM((_CH, CA), f32),
                       pltpu.VMEM((_CH, CP), f32),
                       pltpu.SemaphoreType.DMA, pltpu.SemaphoreType.DMA,
                       pltpu.SemaphoreType.DMA],
    )
    return fn(idx_flat, k, v, p2)


_BQ = 32
_BR = _BQ * KNN


def _stage_c_body(q_ref, g_ref, gate_ref, kg_ref, vg_ref, pg_ref,
                  wb_ref, wo_ref, out_ref):
    f32 = jnp.float32
    pg = pg_ref[...]
    wb = wb_ref[...]
    bias_t = lax.dot_general(wb, pg, (((1,), (1,)), ((), ())),
                             preferred_element_type=f32)
    qb = q_ref[...]
    gb = g_ref[...]
    rows = lax.broadcasted_iota(jnp.int32, (_BQ, _BR), 0)
    cols = lax.broadcasted_iota(jnp.int32, (_BQ, _BR), 1)
    mask = (cols // KNN) == rows
    kg = kg_ref[...]
    vg = vg_ref[...]
    outs = []
    for h in range(NH):
        sl = slice(h * DH, (h + 1) * DH)
        logits = lax.dot_general(qb[:, sl], kg[:, sl],
                                 (((1,), (1,)), ((), ())),
                                 preferred_element_type=f32)
        lgt = jnp.where(mask, logits + bias_t[h:h + 1, :], -1e30)
        m = jnp.max(lgt, axis=1, keepdims=True)
        pr = jnp.exp(lgt - m)
        pr = pr / jnp.sum(pr, axis=1, keepdims=True)
        oh = jnp.dot(pr, vg[:, sl], preferred_element_type=f32)
        outs.append(oh * gb[:, sl])
    o = jnp.concatenate(outs, axis=1)
    o = jnp.dot(o, wo_ref[...], preferred_element_type=f32)
    out_ref[...] = o * gate_ref[...]


def _stage_c(q, g, gate, kg, vg, pg, wb, woT):
    f32 = jnp.float32
    qrow = pl.BlockSpec((_BQ, CA), lambda i: (i, 0))
    grow = pl.BlockSpec((_BR, CA), lambda i: (i, 0))
    prow = pl.BlockSpec((_BR, CP), lambda i: (i, 0))

    def full(a):
        return pl.BlockSpec(a.shape, lambda i: tuple(0 for _ in a.shape))

    return pl.pallas_call(
        _stage_c_body,
        grid=(L // _BQ,),
        in_specs=[qrow, qrow, qrow, grow, grow, prow, full(wb), full(woT)],
        out_specs=qrow,
        out_shape=jax.ShapeDtypeStruct((L, CA), f32),
    )(q, g, gate, kg, vg, pg, wb, woT)


def kernel(Q_L, C_L, P_LL, indices, Wq, Wk, Wv, Wg, Wb, Wo, wq_rms, wk_rms,
           ln_s_w, W_ada_g, b_ada_g, W_ada_lin, W_out_g, b_out_g):
    qL = Q_L[0]
    cL = C_L[0]
    p2 = P_LL.reshape(L * L, CP)
    idx_flat = indices.reshape(E)

    q, k, v, g, gate = _stage_a(
        qL, cL,
        ln_s_w.reshape(1, CS),
        W_ada_g.T, b_ada_g.reshape(1, CA), W_ada_lin.T,
        Wq.T, Wk.T, Wv.T, Wg.T,
        wq_rms.reshape(1, CA), wk_rms.reshape(1, CA),
        W_out_g.T, b_out_g.reshape(1, CA))

    kg, vg, pg = _sc_gather(idx_flat, k, v, p2)

    out = _stage_c(q, g, gate, kg, vg, pg, Wb, Wo.T)
    return out.reshape(1, L, CA)


# speedup vs baseline: 1.7770x; 1.7770x over previous
import jax
import jax.numpy as jnp
from jax import lax
from jax.experimental import pallas as pl
from jax.experimental.pallas import tpu as pltpu
from jax.experimental.pallas import tpu_sc as plsc

L = 2048
CA = 768
CS = 384
CP = 16
NH = 12
DH = 64
KNN = 128
E = L * KNN


_RA = 512


def _stage_a_body(qL_ref, cL_ref, lnsw_ref, wadag_ref, badag_ref, wadal_ref,
                  wq_ref, wk_ref, wv_ref, wg_ref, wqr_ref, wkr_ref,
                  woutg_ref, boutg_ref,
                  q_out, k_out, v_out, g_out, gate_out):
    f32 = jnp.float32
    x = qL_ref[...]
    mu = jnp.mean(x, axis=1, keepdims=True)
    xc = x - mu
    var = jnp.mean(xc * xc, axis=1, keepdims=True)
    a = xc * lax.rsqrt(var + 1e-5)

    c = cL_ref[...]
    cmu = jnp.mean(c, axis=1, keepdims=True)
    cc = c - cmu
    cvar = jnp.mean(cc * cc, axis=1, keepdims=True)
    s = cc * lax.rsqrt(cvar + 1e-5) * lnsw_ref[...]

    adag = jax.nn.sigmoid(
        jnp.dot(s, wadag_ref[...], preferred_element_type=f32) + badag_ref[...])
    a = adag * a + jnp.dot(s, wadal_ref[...], preferred_element_type=f32)

    q = jnp.dot(a, wq_ref[...], preferred_element_type=f32)
    qms = jnp.mean(q * q, axis=1, keepdims=True)
    q = q * lax.rsqrt(qms + 1e-8) * wqr_ref[...]
    q_out[...] = q * (1.0 / (DH ** 0.5))

    k = jnp.dot(a, wk_ref[...], preferred_element_type=f32)
    kms = jnp.mean(k * k, axis=1, keepdims=True)
    k_out[...] = k * lax.rsqrt(kms + 1e-8) * wkr_ref[...]

    v_out[...] = jnp.dot(a, wv_ref[...], preferred_element_type=f32)
    g_out[...] = jax.nn.sigmoid(
        jnp.dot(a, wg_ref[...], preferred_element_type=f32))
    gate_out[...] = jax.nn.sigmoid(
        jnp.dot(c, woutg_ref[...], preferred_element_type=f32) + boutg_ref[...])


def _stage_a(qL, cL, lnsw, wadagT, badag, wadalT, wqT, wkT, wvT, wgT,
             wqr, wkr, woutgT, boutg):
    f32 = jnp.float32

    def row_spec(w):
        return pl.BlockSpec((_RA, w), lambda i: (i, 0))

    def full(a):
        return pl.BlockSpec(a.shape, lambda i: tuple(0 for _ in a.shape))

    outs = [jax.ShapeDtypeStruct((L, CA), f32) for _ in range(5)]
    return pl.pallas_call(
        _stage_a_body,
        grid=(L // _RA,),
        in_specs=[row_spec(CA), row_spec(CS), full(lnsw), full(wadagT),
                  full(badag), full(wadalT), full(wqT), full(wkT), full(wvT),
                  full(wgT), full(wqr), full(wkr), full(woutgT), full(boutg)],
        out_specs=[row_spec(CA)] * 5,
        out_shape=outs,
    )(qL, cL, lnsw, wadagT, badag, wadalT, wqT, wkT, wvT, wgT, wqr, wkr,
      woutgT, boutg)


_CH = 64


def _sc_gather_body(idx_hbm, k_hbm, v_hbm, p_hbm,
                    kg_hbm, vg_hbm, pg_hbm,
                    idx_v, pidx_v, kbuf, vbuf, pbuf, semk, semv, semp):
    info = plsc.get_sparse_core_info()
    nw = info.num_cores * info.num_subcores
    per_w = E // nw
    n_chunks = per_w // _CH
    wid = lax.axis_index("c") * info.num_subcores + lax.axis_index("s")
    base_w = wid * per_w

    def chunk(t, carry):
        base = base_w + t * _CH
        pltpu.sync_copy(idx_hbm.at[pl.ds(base, _CH)], idx_v)
        for j in range(_CH // 16):
            iv = idx_v[pl.ds(j * 16, 16)]
            e0 = base + j * 16
            row = (lax.iota(jnp.int32, 16) + e0) >> 7
            pidx_v[pl.ds(j * 16, 16)] = (row << 11) + iv
        cpk = pltpu.async_copy(k_hbm.at[idx_v], kbuf, semk)
        cpv = pltpu.async_copy(v_hbm.at[idx_v], vbuf, semv)
        cpp = pltpu.async_copy(p_hbm.at[pidx_v], pbuf, semp)
        cpk.wait()
        pltpu.sync_copy(kbuf, kg_hbm.at[pl.ds(base, _CH)])
        cpv.wait()
        pltpu.sync_copy(vbuf, vg_hbm.at[pl.ds(base, _CH)])
        cpp.wait()
        pltpu.sync_copy(pbuf, pg_hbm.at[pl.ds(base, _CH)])
        return carry

    lax.fori_loop(0, n_chunks, chunk, 0)


def _sc_gather(idx_flat, k, v, p2):
    f32 = jnp.float32
    i32 = jnp.int32
    mesh = plsc.VectorSubcoreMesh(core_axis_name="c", subcore_axis_name="s")
    fn = pl.kernel(
        _sc_gather_body,
        out_type=[jax.ShapeDtypeStruct((E, CA), f32),
                  jax.ShapeDtypeStruct((E, CA), f32),
                  jax.ShapeDtypeStruct((E, CP), f32)],
        mesh=mesh,
        scratch_types=[pltpu.VMEM((_CH,), i32), pltpu.VMEM((_CH,), i32),
                       pltpu.VMEM((_CH, CA), f32), pltpu.VMEM((_CH, CA), f32),
                       pltpu.VMEM((_CH, CP), f32),
                       pltpu.SemaphoreType.DMA, pltpu.SemaphoreType.DMA,
                       pltpu.SemaphoreType.DMA],
        compiler_params=pltpu.CompilerParams(use_tc_tiling_on_sc=False),
    )
    return fn(idx_flat, k, v, p2)


_BQ = 32
_BR = _BQ * KNN


def _stage_c_body(q_ref, g_ref, gate_ref, kg_ref, vg_ref, pg_ref,
                  wb_ref, wo_ref, out_ref):
    f32 = jnp.float32
    pg = pg_ref[...]
    wb = wb_ref[...]
    bias_t = lax.dot_general(wb, pg, (((1,), (1,)), ((), ())),
                             preferred_element_type=f32)
    qb = q_ref[...]
    gb = g_ref[...]
    rows = lax.broadcasted_iota(jnp.int32, (_BQ, _BR), 0)
    cols = lax.broadcasted_iota(jnp.int32, (_BQ, _BR), 1)
    mask = (cols // KNN) == rows
    kg = kg_ref[...]
    vg = vg_ref[...]
    outs = []
    for h in range(NH):
        sl = slice(h * DH, (h + 1) * DH)
        logits = lax.dot_general(qb[:, sl], kg[:, sl],
                                 (((1,), (1,)), ((), ())),
                                 preferred_element_type=f32)
        lgt = jnp.where(mask, logits + bias_t[h:h + 1, :], -1e30)
        m = jnp.max(lgt, axis=1, keepdims=True)
        pr = jnp.exp(lgt - m)
        pr = pr / jnp.sum(pr, axis=1, keepdims=True)
        oh = jnp.dot(pr, vg[:, sl], preferred_element_type=f32)
        outs.append(oh * gb[:, sl])
    o = jnp.concatenate(outs, axis=1)
    o = jnp.dot(o, wo_ref[...], preferred_element_type=f32)
    out_ref[...] = o * gate_ref[...]


def _stage_c(q, g, gate, kg, vg, pg, wb, woT):
    f32 = jnp.float32
    qrow = pl.BlockSpec((_BQ, CA), lambda i: (i, 0))
    grow = pl.BlockSpec((_BR, CA), lambda i: (i, 0))
    prow = pl.BlockSpec((_BR, CP), lambda i: (i, 0))

    def full(a):
        return pl.BlockSpec(a.shape, lambda i: tuple(0 for _ in a.shape))

    return pl.pallas_call(
        _stage_c_body,
        grid=(L // _BQ,),
        in_specs=[qrow, qrow, qrow, grow, grow, prow, full(wb), full(woT)],
        out_specs=qrow,
        out_shape=jax.ShapeDtypeStruct((L, CA), f32),
    )(q, g, gate, kg, vg, pg, wb, woT)


def kernel(Q_L, C_L, P_LL, indices, Wq, Wk, Wv, Wg, Wb, Wo, wq_rms, wk_rms,
           ln_s_w, W_ada_g, b_ada_g, W_ada_lin, W_out_g, b_out_g):
    qL = Q_L[0]
    cL = C_L[0]
    p2 = P_LL.reshape(L * L, CP)
    idx_flat = indices.reshape(E)

    q, k, v, g, gate = _stage_a(
        qL, cL,
        ln_s_w.reshape(1, CS),
        W_ada_g.T, b_ada_g.reshape(1, CA), W_ada_lin.T,
        Wq.T, Wk.T, Wv.T, Wg.T,
        wq_rms.reshape(1, CA), wk_rms.reshape(1, CA),
        W_out_g.T, b_out_g.reshape(1, CA))

    kg, vg, pg = _sc_gather(idx_flat, k, v, p2)

    out = _stage_c(q, g, gate, kg, vg, pg, Wb, Wo.T)
    return out.reshape(1, L, CA)


# speedup vs baseline: 2.6033x; 1.4650x over previous
import jax
import jax.numpy as jnp
from jax import lax
from jax.experimental import pallas as pl
from jax.experimental.pallas import tpu as pltpu
from jax.experimental.pallas import tpu_sc as plsc

L = 2048
CA = 768
CS = 384
CP = 16
NH = 12
DH = 64
KNN = 128
E = L * KNN


_RA = 512


def _stage_a_body(qL_ref, cL_ref, lnsw_ref, wadag_ref, badag_ref, wadal_ref,
                  wq_ref, wk_ref, wv_ref, wg_ref, wqr_ref, wkr_ref,
                  woutg_ref, boutg_ref,
                  q_out, k_out, v_out, g_out, gate_out):
    f32 = jnp.float32
    x = qL_ref[...]
    mu = jnp.mean(x, axis=1, keepdims=True)
    xc = x - mu
    var = jnp.mean(xc * xc, axis=1, keepdims=True)
    a = xc * lax.rsqrt(var + 1e-5)

    c = cL_ref[...]
    cmu = jnp.mean(c, axis=1, keepdims=True)
    cc = c - cmu
    cvar = jnp.mean(cc * cc, axis=1, keepdims=True)
    s = cc * lax.rsqrt(cvar + 1e-5) * lnsw_ref[...]

    adag = jax.nn.sigmoid(
        jnp.dot(s, wadag_ref[...], preferred_element_type=f32) + badag_ref[...])
    a = adag * a + jnp.dot(s, wadal_ref[...], preferred_element_type=f32)

    q = jnp.dot(a, wq_ref[...], preferred_element_type=f32)
    qms = jnp.mean(q * q, axis=1, keepdims=True)
    q = q * lax.rsqrt(qms + 1e-8) * wqr_ref[...]
    q_out[...] = q * (1.0 / (DH ** 0.5))

    k = jnp.dot(a, wk_ref[...], preferred_element_type=f32)
    kms = jnp.mean(k * k, axis=1, keepdims=True)
    k_out[...] = k * lax.rsqrt(kms + 1e-8) * wkr_ref[...]

    v_out[...] = jnp.dot(a, wv_ref[...], preferred_element_type=f32)
    g_out[...] = jax.nn.sigmoid(
        jnp.dot(a, wg_ref[...], preferred_element_type=f32))
    gate_out[...] = jax.nn.sigmoid(
        jnp.dot(c, woutg_ref[...], preferred_element_type=f32) + boutg_ref[...])


def _stage_a(qL, cL, lnsw, wadagT, badag, wadalT, wqT, wkT, wvT, wgT,
             wqr, wkr, woutgT, boutg):
    f32 = jnp.float32

    def row_spec(w):
        return pl.BlockSpec((_RA, w), lambda i: (i, 0))

    def full(a):
        return pl.BlockSpec(a.shape, lambda i: tuple(0 for _ in a.shape))

    outs = [jax.ShapeDtypeStruct((L, CA), f32) for _ in range(5)]
    return pl.pallas_call(
        _stage_a_body,
        grid=(L // _RA,),
        in_specs=[row_spec(CA), row_spec(CS), full(lnsw), full(wadagT),
                  full(badag), full(wadalT), full(wqT), full(wkT), full(wvT),
                  full(wgT), full(wqr), full(wkr), full(woutgT), full(boutg)],
        out_specs=[row_spec(CA)] * 5,
        out_shape=outs,
    )(qL, cL, lnsw, wadagT, badag, wadalT, wqT, wkT, wvT, wgT, wqr, wkr,
      woutgT, boutg)


_CH = 64


def _sc_gather_kv_body(idx_hbm, k_hbm, v_hbm, kg_hbm, vg_hbm,
                       idx_v, kbuf, vbuf, semk, semv):
    info = plsc.get_sparse_core_info()
    nw = info.num_cores * info.num_subcores
    per_w = E // nw
    n_chunks = per_w // _CH
    wid = lax.axis_index("c") * info.num_subcores + lax.axis_index("s")
    base_w = wid * per_w

    def chunk(t, carry):
        base = base_w + t * _CH
        pltpu.sync_copy(idx_hbm.at[pl.ds(base, _CH)], idx_v)
        cpk = pltpu.async_copy(k_hbm.at[idx_v], kbuf, semk)
        cpv = pltpu.async_copy(v_hbm.at[idx_v], vbuf, semv)
        cpk.wait()
        pltpu.sync_copy(kbuf, kg_hbm.at[pl.ds(base, _CH)])
        cpv.wait()
        pltpu.sync_copy(vbuf, vg_hbm.at[pl.ds(base, _CH)])
        return carry

    lax.fori_loop(0, n_chunks, chunk, 0)


def _sc_gather_kv(idx_flat, k, v):
    f32 = jnp.float32
    i32 = jnp.int32
    mesh = plsc.VectorSubcoreMesh(core_axis_name="c", subcore_axis_name="s")
    fn = pl.kernel(
        _sc_gather_kv_body,
        out_type=[jax.ShapeDtypeStruct((E, CA), f32),
                  jax.ShapeDtypeStruct((E, CA), f32)],
        mesh=mesh,
        scratch_types=[pltpu.VMEM((_CH,), i32),
                       pltpu.VMEM((_CH, CA), f32), pltpu.VMEM((_CH, CA), f32),
                       pltpu.SemaphoreType.DMA, pltpu.SemaphoreType.DMA],
    )
    return fn(idx_flat, k, v)


_CHP = 256


def _sc_gather_p_body(idx_hbm, p_hbm, pg_hbm, idx_v, pidx_v, pbuf, semp):
    info = plsc.get_sparse_core_info()
    nw = info.num_cores * info.num_subcores
    per_w = E // nw
    n_chunks = per_w // _CHP
    wid = lax.axis_index("c") * info.num_subcores + lax.axis_index("s")
    base_w = wid * per_w

    def chunk(t, carry):
        base = base_w + t * _CHP
        pltpu.sync_copy(idx_hbm.at[pl.ds(base, _CHP)], idx_v)
        for j in range(_CHP // 16):
            iv = idx_v[pl.ds(j * 16, 16)]
            e0 = base + j * 16
            row = (lax.iota(jnp.int32, 16) + e0) >> 7
            pidx_v[pl.ds(j * 16, 16)] = (row << 11) + iv
        cpp = pltpu.async_copy(p_hbm.at[pidx_v], pbuf, semp)
        cpp.wait()
        pltpu.sync_copy(pbuf, pg_hbm.at[pl.ds(base, _CHP)])
        return carry

    lax.fori_loop(0, n_chunks, chunk, 0)


def _sc_gather_p(idx_flat, p2):
    f32 = jnp.float32
    i32 = jnp.int32
    mesh = plsc.VectorSubcoreMesh(core_axis_name="c", subcore_axis_name="s")
    fn = pl.kernel(
        _sc_gather_p_body,
        out_type=jax.ShapeDtypeStruct((E, CP), f32),
        mesh=mesh,
        scratch_types=[pltpu.VMEM((_CHP,), i32), pltpu.VMEM((_CHP,), i32),
                       pltpu.VMEM((_CHP, CP), f32),
                       pltpu.SemaphoreType.DMA],
        compiler_params=pltpu.CompilerParams(use_tc_tiling_on_sc=False),
    )
    return fn(idx_flat, p2)


_BQ = 32
_BR = _BQ * KNN


def _stage_c_body(q_ref, g_ref, gate_ref, kg_ref, vg_ref, pg_ref,
                  wb_ref, wo_ref, out_ref):
    f32 = jnp.float32
    pg = pg_ref[...]
    wb = wb_ref[...]
    bias_t = lax.dot_general(wb, pg, (((1,), (1,)), ((), ())),
                             preferred_element_type=f32)
    qb = q_ref[...]
    gb = g_ref[...]
    rows = lax.broadcasted_iota(jnp.int32, (_BQ, _BR), 0)
    cols = lax.broadcasted_iota(jnp.int32, (_BQ, _BR), 1)
    mask = (cols // KNN) == rows
    kg = kg_ref[...]
    vg = vg_ref[...]
    outs = []
    for h in range(NH):
        sl = slice(h * DH, (h + 1) * DH)
        logits = lax.dot_general(qb[:, sl], kg[:, sl],
                                 (((1,), (1,)), ((), ())),
                                 preferred_element_type=f32)
        lgt = jnp.where(mask, logits + bias_t[h:h + 1, :], -1e30)
        m = jnp.max(lgt, axis=1, keepdims=True)
        pr = jnp.exp(lgt - m)
        pr = pr / jnp.sum(pr, axis=1, keepdims=True)
        oh = jnp.dot(pr, vg[:, sl], preferred_element_type=f32)
        outs.append(oh * gb[:, sl])
    o = jnp.concatenate(outs, axis=1)
    o = jnp.dot(o, wo_ref[...], preferred_element_type=f32)
    out_ref[...] = o * gate_ref[...]


def _stage_c(q, g, gate, kg, vg, pg, wb, woT):
    f32 = jnp.float32
    qrow = pl.BlockSpec((_BQ, CA), lambda i: (i, 0))
    grow = pl.BlockSpec((_BR, CA), lambda i: (i, 0))
    prow = pl.BlockSpec((_BR, CP), lambda i: (i, 0))

    def full(a):
        return pl.BlockSpec(a.shape, lambda i: tuple(0 for _ in a.shape))

    return pl.pallas_call(
        _stage_c_body,
        grid=(L // _BQ,),
        in_specs=[qrow, qrow, qrow, grow, grow, prow, full(wb), full(woT)],
        out_specs=qrow,
        out_shape=jax.ShapeDtypeStruct((L, CA), f32),
    )(q, g, gate, kg, vg, pg, wb, woT)


def kernel(Q_L, C_L, P_LL, indices, Wq, Wk, Wv, Wg, Wb, Wo, wq_rms, wk_rms,
           ln_s_w, W_ada_g, b_ada_g, W_ada_lin, W_out_g, b_out_g):
    qL = Q_L[0]
    cL = C_L[0]
    p2 = P_LL.reshape(L * L, CP)
    idx_flat = indices.reshape(E)

    q, k, v, g, gate = _stage_a(
        qL, cL,
        ln_s_w.reshape(1, CS),
        W_ada_g.T, b_ada_g.reshape(1, CA), W_ada_lin.T,
        Wq.T, Wk.T, Wv.T, Wg.T,
        wq_rms.reshape(1, CA), wk_rms.reshape(1, CA),
        W_out_g.T, b_out_g.reshape(1, CA))

    kg, vg = _sc_gather_kv(idx_flat, k, v)
    pg = _sc_gather_p(idx_flat, p2)

    out = _stage_c(q, g, gate, kg, vg, pg, Wb, Wo.T)
    return out.reshape(1, L, CA)
